# Initial kernel scaffold; baseline (speedup 1.0000x reference)
import jax
import jax.numpy as jnp
from jax.experimental import pallas as pl


def kernel(x, edge_index, W1, b1, W2, b2):
    raise NotImplementedError("write your pallas kernel here")


# speedup vs baseline: 5.8371x; 5.8371x over previous
import functools

import jax
import jax.numpy as jnp
from jax import lax
from jax.experimental import pallas as pl
from jax.experimental.pallas import tpu as pltpu
from jax.experimental.pallas import tpu_sc as plsc

N_NODES = 10000
NPAD = 10112
NC, NS = 2, 16
NW = NC * NS
ROWS_PER_TILE = NPAD // NS
CHUNK = 128
E_EDGES = 320000
EPW = 10112
NCH = EPW // CHUNK
EPAD = EPW * NW

_MESH = plsc.VectorSubcoreMesh(core_axis_name="c", subcore_axis_name="s")
_SC_PARAMS = pltpu.CompilerParams(use_tc_tiling_on_sc=False)


def _agg1_body(y_hbm, src_hbm, dst_hbm, z128_hbm, z16_hbm, ones_hbm,
               acc_hbm, deg_hbm,
               src_v, dst_v, rows_v, ones_v, a_sh, d_sh, sem):
    c = lax.axis_index("c")
    s = lax.axis_index("s")
    w = c * NS + s
    r0 = s * ROWS_PER_TILE
    pltpu.sync_copy(z128_hbm.at[pl.ds(r0, ROWS_PER_TILE)],
                    a_sh.at[pl.ds(r0, ROWS_PER_TILE)])
    pltpu.sync_copy(z16_hbm.at[pl.ds(r0, ROWS_PER_TILE)],
                    d_sh.at[pl.ds(r0, ROWS_PER_TILE)])
    pltpu.sync_copy(ones_hbm, ones_v)
    pltpu.sync_copy(src_hbm.at[pl.ds(w * NCH, NCH)], src_v)
    pltpu.sync_copy(dst_hbm.at[pl.ds(w * NCH, NCH)], dst_v)
    plsc.subcore_barrier()

    def body(j, carry):
        pltpu.async_copy(y_hbm.at[src_v.at[j]], rows_v, sem).wait()
        pltpu.sync_copy(rows_v, a_sh.at[dst_v.at[j]], add=True)
        pltpu.sync_copy(ones_v, d_sh.at[dst_v.at[j]], add=True)
        return carry

    lax.fori_loop(0, NCH, body, 0)
    plsc.subcore_barrier()
    out_r = c * NPAD + r0
    pltpu.sync_copy(a_sh.at[pl.ds(r0, ROWS_PER_TILE)],
                    acc_hbm.at[pl.ds(out_r, ROWS_PER_TILE)])
    pltpu.sync_copy(d_sh.at[pl.ds(r0, ROWS_PER_TILE)],
                    deg_hbm.at[pl.ds(out_r, ROWS_PER_TILE)])


_agg1 = pl.kernel(
    _agg1_body,
    mesh=_MESH,
    compiler_params=_SC_PARAMS,
    out_type=[
        jax.ShapeDtypeStruct((NC * NPAD, 128), jnp.float32),
        jax.ShapeDtypeStruct((NC * NPAD, 16), jnp.float32),
    ],
    scratch_types=[
        pltpu.VMEM((NCH, CHUNK), jnp.int32),
        pltpu.VMEM((NCH, CHUNK), jnp.int32),
        pltpu.VMEM((CHUNK, 128), jnp.float32),
        pltpu.VMEM((CHUNK, 16), jnp.float32),
        pltpu.VMEM_SHARED((NPAD, 128), jnp.float32),
        pltpu.VMEM_SHARED((NPAD, 16), jnp.float32),
        pltpu.SemaphoreType.DMA,
    ],
)


def _agg2_body(y_hbm, src_hbm, dst_hbm, z64_hbm,
               acc_hbm,
               src_v, dst_v, rows_v, a_sh, sem):
    c = lax.axis_index("c")
    s = lax.axis_index("s")
    w = c * NS + s
    r0 = s * ROWS_PER_TILE
    pltpu.sync_copy(z64_hbm.at[pl.ds(r0, ROWS_PER_TILE)],
                    a_sh.at[pl.ds(r0, ROWS_PER_TILE)])
    pltpu.sync_copy(src_hbm.at[pl.ds(w * NCH, NCH)], src_v)
    pltpu.sync_copy(dst_hbm.at[pl.ds(w * NCH, NCH)], dst_v)
    plsc.subcore_barrier()

    def body(j, carry):
        pltpu.async_copy(y_hbm.at[src_v.at[j]], rows_v, sem).wait()
        pltpu.sync_copy(rows_v, a_sh.at[dst_v.at[j]], add=True)
        return carry

    lax.fori_loop(0, NCH, body, 0)
    plsc.subcore_barrier()
    out_r = c * NPAD + r0
    pltpu.sync_copy(a_sh.at[pl.ds(r0, ROWS_PER_TILE)],
                    acc_hbm.at[pl.ds(out_r, ROWS_PER_TILE)])


_agg2 = pl.kernel(
    _agg2_body,
    mesh=_MESH,
    compiler_params=_SC_PARAMS,
    out_type=jax.ShapeDtypeStruct((NC * NPAD, 64), jnp.float32),
    scratch_types=[
        pltpu.VMEM((NCH, CHUNK), jnp.int32),
        pltpu.VMEM((NCH, CHUNK), jnp.int32),
        pltpu.VMEM((CHUNK, 64), jnp.float32),
        pltpu.VMEM_SHARED((NPAD, 64), jnp.float32),
        pltpu.SemaphoreType.DMA,
    ],
)


def _mm_body(x_ref, w_ref, o_ref):
    o_ref[...] = jnp.dot(x_ref[...], w_ref[...],
                         preferred_element_type=jnp.float32)


def _mm(x, w):
    return pl.pallas_call(
        _mm_body,
        out_shape=jax.ShapeDtypeStruct((x.shape[0], w.shape[1]), jnp.float32),
    )(x, w)


def _mid_body(p0, p1, y, d0, d1, b1r, w2, o_ref):
    d = d0[:, 0:1] + d1[:, 0:1]
    inv = 1.0 / (d + 1.0)
    h = jnp.maximum((p0[...] + p1[...] + y[...]) * inv + b1r[...], 0.0)
    o_ref[...] = jnp.dot(h, w2[...], preferred_element_type=jnp.float32)


def _mid(p0, p1, y, d0, d1, b1r, w2):
    return pl.pallas_call(
        _mid_body,
        out_shape=jax.ShapeDtypeStruct((y.shape[0], w2.shape[1]), jnp.float32),
    )(p0, p1, y, d0, d1, b1r, w2)


def _fin_body(q0, q1, y, d0, d1, b2r, o_ref):
    d = d0[:, 0:1] + d1[:, 0:1]
    inv = 1.0 / (d + 1.0)
    o_ref[...] = (q0[...] + q1[...] + y[...]) * inv + b2r[...]


def _fin(q0, q1, y, d0, d1, b2r):
    return pl.pallas_call(
        _fin_body,
        out_shape=jax.ShapeDtypeStruct(y.shape, jnp.float32),
    )(q0, q1, y, d0, d1, b2r)


def kernel(x, edge_index, W1, b1, W2, b2):
    n = x.shape[0]
    pad = EPAD - edge_index.shape[1]
    src_p = jnp.concatenate(
        [edge_index[0], jnp.zeros((pad,), jnp.int32)]).reshape(NW * NCH, CHUNK)
    dst_p = jnp.concatenate(
        [edge_index[1], jnp.full((pad,), n, jnp.int32)]).reshape(NW * NCH, CHUNK)
    z128 = jnp.zeros((NPAD, 128), jnp.float32)
    z64 = jnp.zeros((NPAD, 64), jnp.float32)
    z16 = jnp.zeros((NPAD, 16), jnp.float32)
    onesb = jnp.ones((CHUNK, 16), jnp.float32)

    y1 = _mm(x, W1)
    acc1, deg = _agg1(y1, src_p, dst_p, z128, z16, onesb)
    p0, p1 = acc1[:n], acc1[NPAD:NPAD + n]
    d0, d1 = deg[:n], deg[NPAD:NPAD + n]
    y2 = _mid(p0, p1, y1, d0, d1, b1.reshape(1, -1), W2)
    acc2 = _agg2(y2, src_p, dst_p, z64)
    q0, q1 = acc2[:n], acc2[NPAD:NPAD + n]
    return _fin(q0, q1, y2, d0, d1, b2.reshape(1, -1))
